# Initial kernel scaffold; baseline (speedup 1.0000x reference)
import jax
import jax.numpy as jnp
from jax.experimental import pallas as pl


def kernel(x, W, b, gamma, beta):
    raise NotImplementedError("write your pallas kernel here")


# speedup vs baseline: 4.1002x; 4.1002x over previous
import functools

import jax
import jax.numpy as jnp
from jax import lax
from jax.experimental import pallas as pl
from jax.experimental.pallas import tpu as pltpu
from jax.experimental.pallas import tpu_sc as plsc

_B, _C, _N, _OUT, _K = 4, 64, 4096, 64, 20
_R = 512
_KP = 24
_NW = 32
_G = _B * _N
_CH = _G // _NW
_SUB = 32
_NSUB = _CH // _SUB


def _stage1_kernel(xb_ref, xr_ref, w_ref, bias_ref,
                   u_ref, vt_ref, idx_ref):
    xb = xb_ref[0]
    xr = xr_ref[0]
    W = w_ref[...]
    wc = W[:, :_C]
    wd = W[:, _C:]
    f32 = jnp.float32
    vt = jax.lax.dot_general(xr, wd, (((0,), (1,)), ((), ())),
                             preferred_element_type=f32)
    vt_ref[0] = jnp.concatenate([vt, vt * vt], axis=1)
    u_ref[0] = jax.lax.dot_general(xr, wc - wd, (((0,), (1,)), ((), ())),
                                   preferred_element_type=f32) \
        + bias_ref[0][None, :]
    inner = jax.lax.dot_general(xr, xb, (((0,), (0,)), ((), ())),
                                preferred_element_type=f32)
    xx = jnp.sum(xb * xb, axis=0)
    d = xx[None, :] - 2.0 * inner
    iota = jax.lax.broadcasted_iota(jnp.int32, (_R, _N), 1)
    kiota = jax.lax.broadcasted_iota(jnp.int32, (_R, _KP), 1)
    base = pl.program_id(0) * _N
    inf = f32(jnp.inf)

    def body(k, carry):
        d, ia = carry
        rm = jnp.min(d, axis=1, keepdims=True)
        cand = jnp.where(d == rm, iota, _N)
        mi = jnp.min(cand, axis=1, keepdims=True)
        d = jnp.where(iota == mi, inf, d)
        ia = jnp.where(kiota == k, mi + base, ia)
        return d, ia

    _, ia = jax.lax.fori_loop(
        0, _K, body, (d, jnp.zeros((_R, _KP), jnp.int32)))
    idx_ref[0] = ia


def _sc_gather(vt_hbm, idx_hbm, u_hbm,
               mx_hbm, mn_hbm, accw_hbm,
               idx_b, rows_b, u_b, mx_b, mn_b, acc_b, sem):
    f32 = jnp.float32
    kf = f32(_K)
    wid = lax.axis_index("s") * 2 + lax.axis_index("c")
    base = wid * _CH
    zero = jnp.zeros((16,), f32)

    def sub_body(sb, acc):
        g0 = base + sb * _SUB
        pltpu.sync_copy(idx_hbm.at[pl.ds(g0 * _KP, _SUB * _KP)], idx_b)
        pltpu.sync_copy(u_hbm.at[pl.ds(g0, _SUB)], u_b)
        pltpu.async_copy(vt_hbm.at[idx_b], rows_b, sem).wait()

        def node_body(j, acc):
            a = list(acc)
            jb = j * _KP
            for c in range(4):
                sl = pl.ds(c * 16, 16)
                u = u_b[j, sl]
                s = zero
                q = zero
                mx = jnp.full((16,), -jnp.inf, f32)
                mn = jnp.full((16,), jnp.inf, f32)
                sl2 = pl.ds(_OUT + c * 16, 16)
                for k in range(_K):
                    vv = rows_b[jb + k, sl]
                    s = s + vv
                    q = q + rows_b[jb + k, sl2]
                    mx = jnp.maximum(mx, vv)
                    mn = jnp.minimum(mn, vv)
                mx_b[j, sl] = mx
                mn_b[j, sl] = mn
                a[c] = a[c] + kf * u + s
                a[4 + c] = a[4 + c] + kf * u * u + 2.0 * u * s + q
            return tuple(a)

        acc = lax.fori_loop(0, _SUB, node_body, acc)
        pltpu.sync_copy(mx_b, mx_hbm.at[pl.ds(g0, _SUB)])
        pltpu.sync_copy(mn_b, mn_hbm.at[pl.ds(g0, _SUB)])
        return acc

    acc0 = tuple(zero for _ in range(8))
    acc = lax.fori_loop(0, _NSUB, sub_body, acc0)
    for r in range(2, 8):
        for c in range(4):
            acc_b[r, pl.ds(c * 16, 16)] = zero
    for c in range(4):
        acc_b[0, pl.ds(c * 16, 16)] = acc[c]
        acc_b[1, pl.ds(c * 16, 16)] = acc[4 + c]
    pltpu.sync_copy(acc_b, accw_hbm.at[wid])


def _stage2_kernel(u_ref, mx_ref, mn_ref, acc_ref, g_ref, be_ref, o_ref):
    u = u_ref[0]
    mx = mx_ref[0]
    mn = mn_ref[0]
    acc = jnp.sum(acc_ref[...], axis=0)
    cnt = jnp.float32(_G * _K)
    mean = acc[0, :] / cnt
    var = acc[1, :] / cnt - mean * mean
    rstd = jax.lax.rsqrt(var + 1e-5)
    gamma = g_ref[0]
    beta = be_ref[0]
    a = gamma * rstd
    c = beta - a * mean
    choose = jnp.where((gamma >= 0.0)[None, :], mx, mn)
    o_ref[0] = jnp.maximum(a[None, :] * (u + choose) + c[None, :], 0.0)


def kernel(x, W, b, gamma, beta):
    f32 = jnp.float32
    b2 = b.reshape(1, _OUT).astype(f32)
    g2 = gamma.reshape(1, _OUT).astype(f32)
    be2 = beta.reshape(1, _OUT).astype(f32)
    nb = _N // _R
    row_block = pl.BlockSpec((1, _R, _OUT), lambda b_, i: (b_, i, 0))
    u, vt, idx = pl.pallas_call(
        _stage1_kernel,
        grid=(_B, nb),
        in_specs=[
            pl.BlockSpec((1, _C, _N), lambda b_, i: (b_, 0, 0)),
            pl.BlockSpec((1, _C, _R), lambda b_, i: (b_, 0, i)),
            pl.BlockSpec((_OUT, 2 * _C), lambda b_, i: (0, 0)),
            pl.BlockSpec((1, _OUT), lambda b_, i: (0, 0)),
        ],
        out_specs=[row_block,
                   pl.BlockSpec((1, _R, 2 * _OUT), lambda b_, i: (b_, i, 0)),
                   pl.BlockSpec((1, _R, _KP), lambda b_, i: (b_, i, 0))],
        out_shape=[
            jax.ShapeDtypeStruct((_B, _N, _OUT), f32),
            jax.ShapeDtypeStruct((_B, _N, 2 * _OUT), f32),
            jax.ShapeDtypeStruct((_B, _N, _KP), jnp.int32),
        ],
    )(x, x, W, b2)

    sc = functools.partial(
        pl.kernel,
        mesh=plsc.VectorSubcoreMesh(core_axis_name="c", subcore_axis_name="s"),
        out_type=[
            jax.ShapeDtypeStruct((_G, _OUT), f32),
            jax.ShapeDtypeStruct((_G, _OUT), f32),
            jax.ShapeDtypeStruct((_NW, 8, _OUT), f32),
        ],
        scratch_types=[
            pltpu.VMEM((_SUB * _KP,), jnp.int32),
            pltpu.VMEM((_SUB * _KP, 2 * _OUT), f32),
            pltpu.VMEM((_SUB, _OUT), f32),
            pltpu.VMEM((_SUB, _OUT), f32),
            pltpu.VMEM((_SUB, _OUT), f32),
            pltpu.VMEM((8, _OUT), f32),
            pltpu.SemaphoreType.DMA,
        ],
    )(_sc_gather)
    mx, mn, accw = sc(vt.reshape(_G, 2 * _OUT), idx.reshape(_G * _KP),
                      u.reshape(_G, _OUT))

    out = pl.pallas_call(
        _stage2_kernel,
        grid=(_B, nb),
        in_specs=[
            row_block, row_block, row_block,
            pl.BlockSpec((_NW, 8, _OUT), lambda b_, i: (0, 0, 0)),
            pl.BlockSpec((1, _OUT), lambda b_, i: (0, 0)),
            pl.BlockSpec((1, _OUT), lambda b_, i: (0, 0)),
        ],
        out_specs=row_block,
        out_shape=jax.ShapeDtypeStruct((_B, _N, _OUT), f32),
    )(u, mx.reshape(_B, _N, _OUT), mn.reshape(_B, _N, _OUT),
      accw, g2, be2)
    return jnp.transpose(out, (0, 2, 1))


# speedup vs baseline: 8.1010x; 1.9758x over previous
import functools

import jax
import jax.numpy as jnp
from jax import lax
from jax.experimental import pallas as pl
from jax.experimental.pallas import tpu as pltpu
from jax.experimental.pallas import tpu_sc as plsc

_B, _C, _N, _OUT, _K = 4, 64, 4096, 64, 20
_R = 512
_KP = 20
_NW = 32
_G = _B * _N
_CH = _G // _NW
_SUB = 16
_NSUB = _CH // _SUB


def _stage1_kernel(xb_ref, xr_ref, w_ref, bias_ref,
                   u_ref, vt_ref, idx_ref):
    xb = xb_ref[0]
    xr = xr_ref[0]
    W = w_ref[...]
    wc = W[:, :_C]
    wd = W[:, _C:]
    f32 = jnp.float32
    vt = jax.lax.dot_general(xr, wd, (((0,), (1,)), ((), ())),
                             preferred_element_type=f32)
    vt_ref[0] = jnp.concatenate([vt, vt * vt], axis=1)
    u_ref[0] = jax.lax.dot_general(xr, wc - wd, (((0,), (1,)), ((), ())),
                                   preferred_element_type=f32) \
        + bias_ref[0][None, :]
    inner = jax.lax.dot_general(xr, xb, (((0,), (0,)), ((), ())),
                                preferred_element_type=f32)
    xx = jnp.sum(xb * xb, axis=0)
    d = xx[None, :] - 2.0 * inner
    iota = jax.lax.broadcasted_iota(jnp.int32, (_R, _N), 1)
    kiota = jax.lax.broadcasted_iota(jnp.int32, (_R, _KP), 1)
    base = pl.program_id(0) * _N
    inf = f32(jnp.inf)

    def body(k, carry):
        d, ia = carry
        rm = jnp.min(d, axis=1, keepdims=True)
        cand = jnp.where(d == rm, iota, _N)
        mi = jnp.min(cand, axis=1, keepdims=True)
        d = jnp.where(iota == mi, inf, d)
        ia = jnp.where(kiota == k, mi + base, ia)
        return d, ia

    _, ia = jax.lax.fori_loop(
        0, _K, body, (d, jnp.zeros((_R, _KP), jnp.int32)))
    idx_ref[0] = ia


def _sc_gather(vt_hbm, idx_hbm, u_hbm,
               mx_hbm, mn_hbm, accw_hbm,
               idx_b0, idx_b1, rows_b0, rows_b1, u_b0, u_b1,
               mx_b, mn_b, acc_b, sem0, sem1):
    f32 = jnp.float32
    kf = f32(_K)
    wid = lax.axis_index("s") * 2 + lax.axis_index("c")
    base = wid * _CH
    zero = jnp.zeros((16,), f32)
    ninf = jnp.full((16,), -jnp.inf, f32)
    pinf = jnp.full((16,), jnp.inf, f32)

    def issue(sb, idx_s, u_s, rows_s, sem):
        g0 = base + sb * _SUB
        pltpu.sync_copy(idx_hbm.at[pl.ds(g0 * _KP, _SUB * _KP)], idx_s)
        pltpu.sync_copy(u_hbm.at[pl.ds(g0, _SUB)], u_s)
        pltpu.async_copy(vt_hbm.at[idx_s], rows_s, sem)

    def chunk(sb, rows_s, u_s, acc):
        def pair_body(jp, acc):
            a = list(acc)
            js = (2 * jp, 2 * jp + 1)
            st = [[None] * 4 for _ in range(2)]
            for t in range(2):
                for c in range(4):
                    st[t][c] = [zero, zero, ninf, pinf]
            for k in range(_K):
                for t in range(2):
                    jb = js[t] * _KP + k
                    for c in range(4):
                        vv = rows_s[jb, pl.ds(c * 16, 16)]
                        s_, q_, mx_, mn_ = st[t][c]
                        st[t][c] = [s_ + vv, q_ + vv * vv,
                                    jnp.maximum(mx_, vv),
                                    jnp.minimum(mn_, vv)]
            for t in range(2):
                for c in range(4):
                    sl = pl.ds(c * 16, 16)
                    s_, q_, mx_, mn_ = st[t][c]
                    mx_b[js[t], sl] = mx_
                    mn_b[js[t], sl] = mn_
                    u = u_s[js[t], sl]
                    a[c] = a[c] + kf * u + s_
                    a[4 + c] = a[4 + c] + kf * u * u + 2.0 * u * s_ + q_
            return tuple(a)

        acc = lax.fori_loop(0, _SUB // 2, pair_body, acc)
        g0 = base + sb * _SUB
        pltpu.sync_copy(mx_b, mx_hbm.at[pl.ds(g0, _SUB)])
        pltpu.sync_copy(mn_b, mn_hbm.at[pl.ds(g0, _SUB)])
        return acc

    issue(0, idx_b0, u_b0, rows_b0, sem0)

    def loop(sb2, acc):
        sb = 2 * sb2
        issue(sb + 1, idx_b1, u_b1, rows_b1, sem1)
        pltpu.make_async_copy(vt_hbm.at[idx_b0], rows_b0, sem0).wait()
        acc = chunk(sb, rows_b0, u_b0, acc)

        @pl.when(sb2 < _NSUB // 2 - 1)
        def _():
            issue(sb + 2, idx_b0, u_b0, rows_b0, sem0)

        pltpu.make_async_copy(vt_hbm.at[idx_b1], rows_b1, sem1).wait()
        acc = chunk(sb + 1, rows_b1, u_b1, acc)
        return acc

    acc = lax.fori_loop(0, _NSUB // 2, loop, tuple(zero for _ in range(8)))
    for r in range(2, 8):
        for c in range(4):
            acc_b[r, pl.ds(c * 16, 16)] = zero
    for c in range(4):
        acc_b[0, pl.ds(c * 16, 16)] = acc[c]
        acc_b[1, pl.ds(c * 16, 16)] = acc[4 + c]
    pltpu.sync_copy(acc_b, accw_hbm.at[wid])


def _stage2_kernel(u_ref, mx_ref, mn_ref, acc_ref, g_ref, be_ref, o_ref):
    u = u_ref[0]
    mx = mx_ref[0]
    mn = mn_ref[0]
    acc = jnp.sum(acc_ref[...], axis=0)
    cnt = jnp.float32(_G * _K)
    mean = acc[0, :] / cnt
    var = acc[1, :] / cnt - mean * mean
    rstd = jax.lax.rsqrt(var + 1e-5)
    gamma = g_ref[0]
    beta = be_ref[0]
    a = gamma * rstd
    c = beta - a * mean
    choose = jnp.where((gamma >= 0.0)[None, :], mx, mn)
    o_ref[0] = jnp.maximum(a[None, :] * (u + choose) + c[None, :], 0.0)


def kernel(x, W, b, gamma, beta):
    f32 = jnp.float32
    b2 = b.reshape(1, _OUT).astype(f32)
    g2 = gamma.reshape(1, _OUT).astype(f32)
    be2 = beta.reshape(1, _OUT).astype(f32)
    nb = _N // _R
    row_block = pl.BlockSpec((1, _R, _OUT), lambda b_, i: (b_, i, 0))
    u, vt, idx = pl.pallas_call(
        _stage1_kernel,
        grid=(_B, nb),
        in_specs=[
            pl.BlockSpec((1, _C, _N), lambda b_, i: (b_, 0, 0)),
            pl.BlockSpec((1, _C, _R), lambda b_, i: (b_, 0, i)),
            pl.BlockSpec((_OUT, 2 * _C), lambda b_, i: (0, 0)),
            pl.BlockSpec((1, _OUT), lambda b_, i: (0, 0)),
        ],
        out_specs=[row_block,
                   pl.BlockSpec((1, _R, 2 * _OUT), lambda b_, i: (b_, i, 0)),
                   pl.BlockSpec((1, _R, _KP), lambda b_, i: (b_, i, 0))],
        out_shape=[
            jax.ShapeDtypeStruct((_B, _N, _OUT), f32),
            jax.ShapeDtypeStruct((_B, _N, 2 * _OUT), f32),
            jax.ShapeDtypeStruct((_B, _N, _KP), jnp.int32),
        ],
    )(x, x, W, b2)

    sc = functools.partial(
        pl.kernel,
        mesh=plsc.VectorSubcoreMesh(core_axis_name="c", subcore_axis_name="s"),
        out_type=[
            jax.ShapeDtypeStruct((_G, _OUT), f32),
            jax.ShapeDtypeStruct((_G, _OUT), f32),
            jax.ShapeDtypeStruct((_NW, 8, _OUT), f32),
        ],
        scratch_types=[
            pltpu.VMEM((_SUB * _KP,), jnp.int32),
            pltpu.VMEM((_SUB * _KP,), jnp.int32),
            pltpu.VMEM((_SUB * _KP, 2 * _OUT), f32),
            pltpu.VMEM((_SUB * _KP, 2 * _OUT), f32),
            pltpu.VMEM((_SUB, _OUT), f32),
            pltpu.VMEM((_SUB, _OUT), f32),
            pltpu.VMEM((_SUB, _OUT), f32),
            pltpu.VMEM((_SUB, _OUT), f32),
            pltpu.VMEM((8, _OUT), f32),
            pltpu.SemaphoreType.DMA,
            pltpu.SemaphoreType.DMA,
        ],
    )(_sc_gather)
    mx, mn, accw = sc(vt.reshape(_G, 2 * _OUT), idx.reshape(_G * _KP),
                      u.reshape(_G, _OUT))

    out = pl.pallas_call(
        _stage2_kernel,
        grid=(_B, nb),
        in_specs=[
            row_block, row_block, row_block,
            pl.BlockSpec((_NW, 8, _OUT), lambda b_, i: (0, 0, 0)),
            pl.BlockSpec((1, _OUT), lambda b_, i: (0, 0)),
            pl.BlockSpec((1, _OUT), lambda b_, i: (0, 0)),
        ],
        out_specs=row_block,
        out_shape=jax.ShapeDtypeStruct((_B, _N, _OUT), f32),
    )(u, mx.reshape(_B, _N, _OUT), mn.reshape(_B, _N, _OUT),
      accw, g2, be2)
    return jnp.transpose(out, (0, 2, 1))


# speedup vs baseline: 8.1513x; 1.0062x over previous
import functools

import jax
import jax.numpy as jnp
from jax import lax
from jax.experimental import pallas as pl
from jax.experimental.pallas import tpu as pltpu
from jax.experimental.pallas import tpu_sc as plsc

_B, _C, _N, _OUT, _K = 4, 64, 4096, 64, 20
_R = 1024
_KP = 20
_NW = 32
_G = _B * _N
_CH = _G // _NW
_SUB = 16
_NSUB = _CH // _SUB


def _stage1_kernel(xb_ref, xr_ref, w_ref, bias_ref,
                   u_ref, vt_ref, idx_ref):
    xb = xb_ref[0]
    xr = xr_ref[0]
    W = w_ref[...]
    wc = W[:, :_C]
    wd = W[:, _C:]
    f32 = jnp.float32
    vt = jax.lax.dot_general(xr, wd, (((0,), (1,)), ((), ())),
                             preferred_element_type=f32)
    vt_ref[0] = jnp.concatenate([vt, vt * vt], axis=1)
    u_ref[0] = jax.lax.dot_general(xr, wc - wd, (((0,), (1,)), ((), ())),
                                   preferred_element_type=f32) \
        + bias_ref[0][None, :]
    inner = jax.lax.dot_general(xr, xb, (((0,), (0,)), ((), ())),
                                preferred_element_type=f32)
    xx = jnp.sum(xb * xb, axis=0)
    d = xx[None, :] - 2.0 * inner
    iota = jax.lax.broadcasted_iota(jnp.int32, (_R, _N), 1)
    kiota = jax.lax.broadcasted_iota(jnp.int32, (_R, _KP), 1)
    base = pl.program_id(0) * _N
    inf = f32(jnp.inf)

    def body(k, carry):
        d, ia = carry
        mi = jnp.argmin(d, axis=1)[:, None]
        d = jnp.where(iota == mi, inf, d)
        ia = jnp.where(kiota == k, mi + base, ia)
        return d, ia

    _, ia = jax.lax.fori_loop(
        0, _K, body, (d, jnp.zeros((_R, _KP), jnp.int32)))
    idx_ref[0] = ia


def _sc_gather(vt_hbm, idx_hbm, u_hbm,
               mx_hbm, mn_hbm, accw_hbm,
               idx_b0, idx_b1, rows_b0, rows_b1, u_b0, u_b1,
               mx_b, mn_b, acc_b, sem0, sem1):
    f32 = jnp.float32
    kf = f32(_K)
    wid = lax.axis_index("s") * 2 + lax.axis_index("c")
    base = wid * _CH
    zero = jnp.zeros((16,), f32)
    ninf = jnp.full((16,), -jnp.inf, f32)
    pinf = jnp.full((16,), jnp.inf, f32)

    def issue(sb, idx_s, u_s, rows_s, sem):
        g0 = base + sb * _SUB
        pltpu.sync_copy(idx_hbm.at[pl.ds(g0 * _KP, _SUB * _KP)], idx_s)
        pltpu.sync_copy(u_hbm.at[pl.ds(g0, _SUB)], u_s)
        pltpu.async_copy(vt_hbm.at[idx_s], rows_s, sem)

    def chunk(sb, rows_s, u_s, acc):
        def pair_body(jp, acc):
            a = list(acc)
            js = (2 * jp, 2 * jp + 1)
            st = [[None] * 4 for _ in range(2)]
            for t in range(2):
                for c in range(4):
                    st[t][c] = [zero, zero, ninf, pinf]
            for k in range(_K):
                for t in range(2):
                    jb = js[t] * _KP + k
                    for c in range(4):
                        vv = rows_s[jb, pl.ds(c * 16, 16)]
                        s_, q_, mx_, mn_ = st[t][c]
                        st[t][c] = [s_ + vv, q_ + vv * vv,
                                    jnp.maximum(mx_, vv),
                                    jnp.minimum(mn_, vv)]
            for t in range(2):
                for c in range(4):
                    sl = pl.ds(c * 16, 16)
                    s_, q_, mx_, mn_ = st[t][c]
                    mx_b[js[t], sl] = mx_
                    mn_b[js[t], sl] = mn_
                    u = u_s[js[t], sl]
                    a[c] = a[c] + kf * u + s_
                    a[4 + c] = a[4 + c] + kf * u * u + 2.0 * u * s_ + q_
            return tuple(a)

        acc = lax.fori_loop(0, _SUB // 2, pair_body, acc)
        g0 = base + sb * _SUB
        pltpu.sync_copy(mx_b, mx_hbm.at[pl.ds(g0, _SUB)])
        pltpu.sync_copy(mn_b, mn_hbm.at[pl.ds(g0, _SUB)])
        return acc

    issue(0, idx_b0, u_b0, rows_b0, sem0)

    def loop(sb2, acc):
        sb = 2 * sb2
        issue(sb + 1, idx_b1, u_b1, rows_b1, sem1)
        pltpu.make_async_copy(vt_hbm.at[idx_b0], rows_b0, sem0).wait()
        acc = chunk(sb, rows_b0, u_b0, acc)

        @pl.when(sb2 < _NSUB // 2 - 1)
        def _():
            issue(sb + 2, idx_b0, u_b0, rows_b0, sem0)

        pltpu.make_async_copy(vt_hbm.at[idx_b1], rows_b1, sem1).wait()
        acc = chunk(sb + 1, rows_b1, u_b1, acc)
        return acc

    acc = lax.fori_loop(0, _NSUB // 2, loop, tuple(zero for _ in range(8)))
    for r in range(2, 8):
        for c in range(4):
            acc_b[r, pl.ds(c * 16, 16)] = zero
    for c in range(4):
        acc_b[0, pl.ds(c * 16, 16)] = acc[c]
        acc_b[1, pl.ds(c * 16, 16)] = acc[4 + c]
    pltpu.sync_copy(acc_b, accw_hbm.at[wid])


def _stage2_kernel(u_ref, mx_ref, mn_ref, acc_ref, g_ref, be_ref, o_ref):
    u = u_ref[0]
    mx = mx_ref[0]
    mn = mn_ref[0]
    acc = jnp.sum(acc_ref[...], axis=0)
    cnt = jnp.float32(_G * _K)
    mean = acc[0, :] / cnt
    var = acc[1, :] / cnt - mean * mean
    rstd = jax.lax.rsqrt(var + 1e-5)
    gamma = g_ref[0]
    beta = be_ref[0]
    a = gamma * rstd
    c = beta - a * mean
    choose = jnp.where((gamma >= 0.0)[None, :], mx, mn)
    o_ref[0] = jnp.maximum(a[None, :] * (u + choose) + c[None, :], 0.0)


def kernel(x, W, b, gamma, beta):
    f32 = jnp.float32
    b2 = b.reshape(1, _OUT).astype(f32)
    g2 = gamma.reshape(1, _OUT).astype(f32)
    be2 = beta.reshape(1, _OUT).astype(f32)
    nb = _N // _R
    row_block = pl.BlockSpec((1, _R, _OUT), lambda b_, i: (b_, i, 0))
    u, vt, idx = pl.pallas_call(
        _stage1_kernel,
        grid=(_B, nb),
        in_specs=[
            pl.BlockSpec((1, _C, _N), lambda b_, i: (b_, 0, 0)),
            pl.BlockSpec((1, _C, _R), lambda b_, i: (b_, 0, i)),
            pl.BlockSpec((_OUT, 2 * _C), lambda b_, i: (0, 0)),
            pl.BlockSpec((1, _OUT), lambda b_, i: (0, 0)),
        ],
        out_specs=[row_block,
                   pl.BlockSpec((1, _R, 2 * _OUT), lambda b_, i: (b_, i, 0)),
                   pl.BlockSpec((1, _R, _KP), lambda b_, i: (b_, i, 0))],
        out_shape=[
            jax.ShapeDtypeStruct((_B, _N, _OUT), f32),
            jax.ShapeDtypeStruct((_B, _N, 2 * _OUT), f32),
            jax.ShapeDtypeStruct((_B, _N, _KP), jnp.int32),
        ],
    )(x, x, W, b2)

    sc = functools.partial(
        pl.kernel,
        mesh=plsc.VectorSubcoreMesh(core_axis_name="c", subcore_axis_name="s"),
        out_type=[
            jax.ShapeDtypeStruct((_G, _OUT), f32),
            jax.ShapeDtypeStruct((_G, _OUT), f32),
            jax.ShapeDtypeStruct((_NW, 8, _OUT), f32),
        ],
        scratch_types=[
            pltpu.VMEM((_SUB * _KP,), jnp.int32),
            pltpu.VMEM((_SUB * _KP,), jnp.int32),
            pltpu.VMEM((_SUB * _KP, 2 * _OUT), f32),
            pltpu.VMEM((_SUB * _KP, 2 * _OUT), f32),
            pltpu.VMEM((_SUB, _OUT), f32),
            pltpu.VMEM((_SUB, _OUT), f32),
            pltpu.VMEM((_SUB, _OUT), f32),
            pltpu.VMEM((_SUB, _OUT), f32),
            pltpu.VMEM((8, _OUT), f32),
            pltpu.SemaphoreType.DMA,
            pltpu.SemaphoreType.DMA,
        ],
    )(_sc_gather)
    mx, mn, accw = sc(vt.reshape(_G, 2 * _OUT), idx.reshape(_G * _KP),
                      u.reshape(_G, _OUT))

    out = pl.pallas_call(
        _stage2_kernel,
        grid=(_B, nb),
        in_specs=[
            row_block, row_block, row_block,
            pl.BlockSpec((_NW, 8, _OUT), lambda b_, i: (0, 0, 0)),
            pl.BlockSpec((1, _OUT), lambda b_, i: (0, 0)),
            pl.BlockSpec((1, _OUT), lambda b_, i: (0, 0)),
        ],
        out_specs=row_block,
        out_shape=jax.ShapeDtypeStruct((_B, _N, _OUT), f32),
    )(u, mx.reshape(_B, _N, _OUT), mn.reshape(_B, _N, _OUT),
      accw, g2, be2)
    return jnp.transpose(out, (0, 2, 1))


# speedup vs baseline: 8.9074x; 1.0928x over previous
import functools

import jax
import jax.numpy as jnp
from jax import lax
from jax.experimental import pallas as pl
from jax.experimental.pallas import tpu as pltpu
from jax.experimental.pallas import tpu_sc as plsc

_B, _C, _N, _OUT, _K = 4, 64, 4096, 64, 20
_R = 1024
_KP = 20
_NW = 32
_G = _B * _N
_CH = _G // _NW
_SUB = 16
_NSUB = _CH // _SUB


def _stage1_kernel(xb_ref, xr_ref, w_ref, bias_ref,
                   u_ref, vt_ref, idx_ref):
    xb = xb_ref[0]
    xr = xr_ref[0]
    W = w_ref[...]
    wc = W[:, :_C]
    wd = W[:, _C:]
    f32 = jnp.float32
    vt = jax.lax.dot_general(xr, wd, (((0,), (1,)), ((), ())),
                             preferred_element_type=f32)
    vt_ref[0] = jnp.concatenate([vt, vt * vt], axis=1)
    u_ref[0] = jax.lax.dot_general(xr, wc - wd, (((0,), (1,)), ((), ())),
                                   preferred_element_type=f32) \
        + bias_ref[0][None, :]
    inner = jax.lax.dot_general(xr, xb, (((0,), (0,)), ((), ())),
                                preferred_element_type=f32)
    xx = jnp.sum(xb * xb, axis=0)
    d = xx[None, :] - 2.0 * inner
    iotac = jax.lax.broadcasted_iota(jnp.int32, (_N, 128), 0).astype(jnp.float32)
    kiota = jax.lax.broadcasted_iota(jnp.int32, (1, _KP), 1)
    base = pl.program_id(0) * _N
    inf = f32(jnp.inf)

    def body(k, carry):
        d, ia = carry
        rm = jnp.min(d, axis=1, keepdims=True)
        oh = (d == rm).astype(f32)
        mi_f = jax.lax.dot_general(oh, iotac, (((1,), (0,)), ((), ())),
                                   preferred_element_type=f32)[:, :1]
        mi = jnp.minimum(mi_f.astype(jnp.int32), _N - 1)
        d = jnp.where(d == rm, inf, d)
        ia = jnp.where(kiota == k, mi + base, ia)
        return d, ia

    _, ia = jax.lax.fori_loop(
        0, _K, body, (d, jnp.zeros((_R, _KP), jnp.int32)))
    idx_ref[0] = ia


def _sc_gather(vt_hbm, idx_hbm, u_hbm,
               mx_hbm, mn_hbm, accw_hbm,
               idx_b0, idx_b1, rows_b0, rows_b1, u_b0, u_b1,
               mx_b, mn_b, acc_b, sem0, sem1):
    f32 = jnp.float32
    kf = f32(_K)
    wid = lax.axis_index("s") * 2 + lax.axis_index("c")
    base = wid * _CH
    zero = jnp.zeros((16,), f32)
    ninf = jnp.full((16,), -jnp.inf, f32)
    pinf = jnp.full((16,), jnp.inf, f32)

    def issue(sb, idx_s, u_s, rows_s, sem):
        g0 = base + sb * _SUB
        pltpu.sync_copy(idx_hbm.at[pl.ds(g0 * _KP, _SUB * _KP)], idx_s)
        pltpu.sync_copy(u_hbm.at[pl.ds(g0, _SUB)], u_s)
        pltpu.async_copy(vt_hbm.at[idx_s], rows_s, sem)

    def chunk(sb, rows_s, u_s, acc):
        def pair_body(jp, acc):
            a = list(acc)
            js = (2 * jp, 2 * jp + 1)
            st = [[None] * 4 for _ in range(2)]
            for t in range(2):
                for c in range(4):
                    st[t][c] = [zero, zero, ninf, pinf]
            for k in range(_K):
                for t in range(2):
                    jb = js[t] * _KP + k
                    for c in range(4):
                        vv = rows_s[jb, pl.ds(c * 16, 16)]
                        s_, q_, mx_, mn_ = st[t][c]
                        st[t][c] = [s_ + vv, q_ + vv * vv,
                                    jnp.maximum(mx_, vv),
                                    jnp.minimum(mn_, vv)]
            for t in range(2):
                for c in range(4):
                    sl = pl.ds(c * 16, 16)
                    s_, q_, mx_, mn_ = st[t][c]
                    mx_b[js[t], sl] = mx_
                    mn_b[js[t], sl] = mn_
                    u = u_s[js[t], sl]
                    a[c] = a[c] + kf * u + s_
                    a[4 + c] = a[4 + c] + kf * u * u + 2.0 * u * s_ + q_
            return tuple(a)

        acc = lax.fori_loop(0, _SUB // 2, pair_body, acc)
        g0 = base + sb * _SUB
        pltpu.sync_copy(mx_b, mx_hbm.at[pl.ds(g0, _SUB)])
        pltpu.sync_copy(mn_b, mn_hbm.at[pl.ds(g0, _SUB)])
        return acc

    issue(0, idx_b0, u_b0, rows_b0, sem0)

    def loop(sb2, acc):
        sb = 2 * sb2
        issue(sb + 1, idx_b1, u_b1, rows_b1, sem1)
        pltpu.make_async_copy(vt_hbm.at[idx_b0], rows_b0, sem0).wait()
        acc = chunk(sb, rows_b0, u_b0, acc)

        @pl.when(sb2 < _NSUB // 2 - 1)
        def _():
            issue(sb + 2, idx_b0, u_b0, rows_b0, sem0)

        pltpu.make_async_copy(vt_hbm.at[idx_b1], rows_b1, sem1).wait()
        acc = chunk(sb + 1, rows_b1, u_b1, acc)
        return acc

    acc = lax.fori_loop(0, _NSUB // 2, loop, tuple(zero for _ in range(8)))
    for r in range(2, 8):
        for c in range(4):
            acc_b[r, pl.ds(c * 16, 16)] = zero
    for c in range(4):
        acc_b[0, pl.ds(c * 16, 16)] = acc[c]
        acc_b[1, pl.ds(c * 16, 16)] = acc[4 + c]
    pltpu.sync_copy(acc_b, accw_hbm.at[wid])


def _stage2_kernel(u_ref, mx_ref, mn_ref, acc_ref, g_ref, be_ref, o_ref):
    u = u_ref[0]
    mx = mx_ref[0]
    mn = mn_ref[0]
    acc = jnp.sum(acc_ref[...], axis=0)
    cnt = jnp.float32(_G * _K)
    mean = acc[0, :] / cnt
    var = acc[1, :] / cnt - mean * mean
    rstd = jax.lax.rsqrt(var + 1e-5)
    gamma = g_ref[0]
    beta = be_ref[0]
    a = gamma * rstd
    c = beta - a * mean
    choose = jnp.where((gamma >= 0.0)[None, :], mx, mn)
    o_ref[0] = jnp.maximum(a[None, :] * (u + choose) + c[None, :], 0.0)


def kernel(x, W, b, gamma, beta):
    f32 = jnp.float32
    b2 = b.reshape(1, _OUT).astype(f32)
    g2 = gamma.reshape(1, _OUT).astype(f32)
    be2 = beta.reshape(1, _OUT).astype(f32)
    nb = _N // _R
    row_block = pl.BlockSpec((1, _R, _OUT), lambda b_, i: (b_, i, 0))
    u, vt, idx = pl.pallas_call(
        _stage1_kernel,
        grid=(_B, nb),
        in_specs=[
            pl.BlockSpec((1, _C, _N), lambda b_, i: (b_, 0, 0)),
            pl.BlockSpec((1, _C, _R), lambda b_, i: (b_, 0, i)),
            pl.BlockSpec((_OUT, 2 * _C), lambda b_, i: (0, 0)),
            pl.BlockSpec((1, _OUT), lambda b_, i: (0, 0)),
        ],
        out_specs=[row_block,
                   pl.BlockSpec((1, _R, 2 * _OUT), lambda b_, i: (b_, i, 0)),
                   pl.BlockSpec((1, _R, _KP), lambda b_, i: (b_, i, 0))],
        out_shape=[
            jax.ShapeDtypeStruct((_B, _N, _OUT), f32),
            jax.ShapeDtypeStruct((_B, _N, 2 * _OUT), f32),
            jax.ShapeDtypeStruct((_B, _N, _KP), jnp.int32),
        ],
    )(x, x, W, b2)

    sc = functools.partial(
        pl.kernel,
        mesh=plsc.VectorSubcoreMesh(core_axis_name="c", subcore_axis_name="s"),
        out_type=[
            jax.ShapeDtypeStruct((_G, _OUT), f32),
            jax.ShapeDtypeStruct((_G, _OUT), f32),
            jax.ShapeDtypeStruct((_NW, 8, _OUT), f32),
        ],
        scratch_types=[
            pltpu.VMEM((_SUB * _KP,), jnp.int32),
            pltpu.VMEM((_SUB * _KP,), jnp.int32),
            pltpu.VMEM((_SUB * _KP, 2 * _OUT), f32),
            pltpu.VMEM((_SUB * _KP, 2 * _OUT), f32),
            pltpu.VMEM((_SUB, _OUT), f32),
            pltpu.VMEM((_SUB, _OUT), f32),
            pltpu.VMEM((_SUB, _OUT), f32),
            pltpu.VMEM((_SUB, _OUT), f32),
            pltpu.VMEM((8, _OUT), f32),
            pltpu.SemaphoreType.DMA,
            pltpu.SemaphoreType.DMA,
        ],
    )(_sc_gather)
    mx, mn, accw = sc(vt.reshape(_G, 2 * _OUT), idx.reshape(_G * _KP),
                      u.reshape(_G, _OUT))

    out = pl.pallas_call(
        _stage2_kernel,
        grid=(_B, nb),
        in_specs=[
            row_block, row_block, row_block,
            pl.BlockSpec((_NW, 8, _OUT), lambda b_, i: (0, 0, 0)),
            pl.BlockSpec((1, _OUT), lambda b_, i: (0, 0)),
            pl.BlockSpec((1, _OUT), lambda b_, i: (0, 0)),
        ],
        out_specs=row_block,
        out_shape=jax.ShapeDtypeStruct((_B, _N, _OUT), f32),
    )(u, mx.reshape(_B, _N, _OUT), mn.reshape(_B, _N, _OUT),
      accw, g2, be2)
    return jnp.transpose(out, (0, 2, 1))
